# baseline (device time: 148030 ns/iter reference)
import jax
import jax.numpy as jnp
from jax import lax
from jax.experimental import pallas as pl
from jax.experimental.pallas import tpu as pltpu

CHUNK = 256


def kernel(x, dest):
    m, n = x.shape
    max_chunks = m // CHUNK

    my_x = lax.axis_index("x")
    keep = dest == my_x
    k = jnp.sum(keep.astype(jnp.int32))
    csum = jnp.cumsum(keep.astype(jnp.int32))
    iota = jnp.arange(m, dtype=jnp.int32)
    slot = jnp.where(keep, csum - 1, k + iota - csum)
    src_of = jnp.zeros((m,), jnp.int32).at[slot].set(iota, unique_indices=True)

    def pack_pairs(a):
        p = a.reshape(m // 2, 2)
        pk = p[:, 0] + p[:, 1] * 4096
        return jnp.concatenate([pk, jnp.zeros((1,), jnp.int32)])

    src_send = lax.dynamic_slice(
        jnp.concatenate([src_of, jnp.zeros((m,), jnp.int32)]), (k,), (m,)
    )
    pk_keep = pack_pairs(src_of)
    pk_send = pack_pairs(src_send)
    counts = jnp.stack([k, m - k]).astype(jnp.int32)

    def body(x_ref, pks_ref, pkk_ref, cnt_ref, out_ref, send_buf,
             send_sems, recv_sems):
        mx = lax.axis_index("x")
        peer = (1 - mx, lax.axis_index("y"), lax.axis_index("z"))
        kk = cnt_ref[0]
        ss = cnt_ref[1]
        rr = ss
        keep_base = jnp.where(mx == 0, 0, rr)
        remote_base = jnp.where(mx == 0, 0, m - ss)

        barrier_sem = pltpu.get_barrier_semaphore()
        pl.semaphore_signal(
            barrier_sem, inc=1, device_id=peer,
            device_id_type=pl.DeviceIdType.MESH,
        )
        pl.semaphore_wait(barrier_sem, 1)

        def rowslice(ref, idx, rows=1):
            return ref.at[pl.ds(pl.multiple_of(idx * n, n), rows * n)]

        def vcopy(dst_ref, dst_row, src_row):
            dst_ref[pl.ds(pl.multiple_of(dst_row * n, n), n)] = (
                x_ref[pl.ds(pl.multiple_of(src_row * n, n), n)]
            )

        n_send = (ss + CHUNK - 1) // CHUNK

        def send_chunk(j, off):
            pltpu.make_async_remote_copy(
                src_ref=rowslice(send_buf, off, CHUNK),
                dst_ref=rowslice(out_ref, remote_base + off, CHUNK),
                send_sem=send_sems.at[j],
                recv_sem=recv_sems.at[j],
                device_id=peer,
                device_id_type=pl.DeviceIdType.MESH,
            ).start()

        half = CHUNK // 2
        np_send = ss // 2

        def comp_pair(t, e):
            nxt = pks_ref[t + 1]
            vcopy(send_buf, 2 * t, e % 4096)
            vcopy(send_buf, 2 * t + 1, e // 4096)
            return nxt

        cur = pks_ref[0]
        for c in range(max_chunks):
            cur = lax.fori_loop(
                jnp.minimum(c * half, np_send),
                jnp.minimum((c + 1) * half, np_send),
                comp_pair,
                cur,
            )

            @pl.when(c + 1 < n_send)
            def _(c=c):
                send_chunk(c, c * CHUNK)

        @pl.when(ss % 2 == 1)
        def _():
            vcopy(send_buf, ss - 1, pks_ref[ss // 2] % 4096)

        for c in range(max_chunks):
            @pl.when(c == n_send - 1)
            def _(c=c):
                send_chunk(c, ss - CHUNK)

        def keep_pair(t, e):
            nxt = pkk_ref[t + 1]
            vcopy(out_ref, keep_base + 2 * t, e % 4096)
            vcopy(out_ref, keep_base + 2 * t + 1, e // 4096)
            return nxt

        lax.fori_loop(0, kk // 2, keep_pair, pkk_ref[0])

        @pl.when(kk % 2 == 1)
        def _():
            vcopy(out_ref, keep_base + kk - 1, pkk_ref[kk // 2] % 4096)

        n_recv = (rr + CHUNK - 1) // CHUNK
        for j in range(max_chunks):
            @pl.when(j < n_recv)
            def _(j=j):
                pltpu.make_async_remote_copy(
                    src_ref=rowslice(send_buf, 0, CHUNK),
                    dst_ref=rowslice(out_ref, 0, CHUNK),
                    send_sem=send_sems.at[j],
                    recv_sem=recv_sems.at[j],
                    device_id=peer,
                    device_id_type=pl.DeviceIdType.MESH,
                ).wait_recv()

        for j in range(max_chunks):
            @pl.when(j < n_send)
            def _(j=j):
                pltpu.make_async_remote_copy(
                    src_ref=rowslice(send_buf, 0, CHUNK),
                    dst_ref=rowslice(out_ref, 0, CHUNK),
                    send_sem=send_sems.at[j],
                    recv_sem=recv_sems.at[j],
                    device_id=peer,
                    device_id_type=pl.DeviceIdType.MESH,
                ).wait_send()

    out_flat = pl.pallas_call(
        body,
        out_shape=jax.ShapeDtypeStruct((m * n,), x.dtype),
        in_specs=[
            pl.BlockSpec(memory_space=pltpu.VMEM),
            pl.BlockSpec(memory_space=pltpu.SMEM),
            pl.BlockSpec(memory_space=pltpu.SMEM),
            pl.BlockSpec(memory_space=pltpu.SMEM),
        ],
        out_specs=pl.BlockSpec(memory_space=pltpu.VMEM),
        scratch_shapes=[
            pltpu.VMEM((m * n,), x.dtype),
            pltpu.SemaphoreType.DMA((m // CHUNK,)),
            pltpu.SemaphoreType.DMA((m // CHUNK,)),
        ],
        compiler_params=pltpu.CompilerParams(collective_id=0),
    )(x.reshape(m * n), pk_send, pk_keep, counts)
    return out_flat.reshape(m, n)


# device time: 144576 ns/iter; 1.0239x vs baseline; 1.0239x over previous
import jax
import jax.numpy as jnp
from jax import lax
from jax.experimental import pallas as pl
from jax.experimental.pallas import tpu as pltpu

CHUNK = 256


def kernel(x, dest):
    m, n = x.shape
    max_chunks = m // CHUNK

    my_x = lax.axis_index("x")
    keep = dest == my_x
    k = jnp.sum(keep.astype(jnp.int32))
    csum = jnp.cumsum(keep.astype(jnp.int32))
    iota = jnp.arange(m, dtype=jnp.int32)
    slot = jnp.where(keep, csum - 1, k + iota - csum)
    src_of = jnp.zeros((m,), jnp.int32).at[slot].set(iota, unique_indices=True)
    counts = jnp.stack([k, m - k]).astype(jnp.int32)

    def body(x_ref, src_ref, cnt_ref, out_ref, send_buf,
             send_sems, recv_sems):
        mx = lax.axis_index("x")
        peer = (1 - mx, lax.axis_index("y"), lax.axis_index("z"))
        kk = cnt_ref[0]
        ss = cnt_ref[1]
        rr = ss
        keep_base = jnp.where(mx == 0, 0, rr)
        remote_base = jnp.where(mx == 0, 0, m - ss)

        barrier_sem = pltpu.get_barrier_semaphore()
        pl.semaphore_signal(
            barrier_sem, inc=1, device_id=peer,
            device_id_type=pl.DeviceIdType.MESH,
        )
        pl.semaphore_wait(barrier_sem, 1)

        def rowslice(ref, idx, rows=1):
            return ref.at[pl.ds(pl.multiple_of(idx * n, n), rows * n)]

        def vcopy(dst_ref, dst_row, src_row):
            dst_ref[pl.ds(pl.multiple_of(dst_row * n, n), n)] = (
                x_ref[pl.ds(pl.multiple_of(src_row * n, n), n)]
            )

        n_send = (ss + CHUNK - 1) // CHUNK

        def send_chunk(j, off):
            pltpu.make_async_remote_copy(
                src_ref=rowslice(send_buf, off, CHUNK),
                dst_ref=rowslice(out_ref, remote_base + off, CHUNK),
                send_sem=send_sems.at[j],
                recv_sem=recv_sems.at[j],
                device_id=peer,
                device_id_type=pl.DeviceIdType.MESH,
            ).start()

        for c in range(max_chunks):
            def comp_body(t, z, c=c):
                vcopy(send_buf, t, src_ref[kk + t])
                return z

            lax.fori_loop(
                jnp.minimum(c * CHUNK, ss),
                jnp.minimum((c + 1) * CHUNK, ss),
                comp_body,
                jnp.int32(0),
            )

            @pl.when(c + 1 < n_send)
            def _(c=c):
                send_chunk(c, c * CHUNK)

        for c in range(max_chunks):
            @pl.when(c == n_send - 1)
            def _(c=c):
                send_chunk(c, ss - CHUNK)

        def keep_body(t, z):
            vcopy(out_ref, keep_base + t, src_ref[t])
            return z

        lax.fori_loop(0, kk, keep_body, jnp.int32(0))

        n_recv = (rr + CHUNK - 1) // CHUNK
        for j in range(max_chunks):
            @pl.when(j < n_recv)
            def _(j=j):
                pltpu.make_async_remote_copy(
                    src_ref=rowslice(send_buf, 0, CHUNK),
                    dst_ref=rowslice(out_ref, 0, CHUNK),
                    send_sem=send_sems.at[j],
                    recv_sem=recv_sems.at[j],
                    device_id=peer,
                    device_id_type=pl.DeviceIdType.MESH,
                ).wait_recv()

        for j in range(max_chunks):
            @pl.when(j < n_send)
            def _(j=j):
                pltpu.make_async_remote_copy(
                    src_ref=rowslice(send_buf, 0, CHUNK),
                    dst_ref=rowslice(out_ref, 0, CHUNK),
                    send_sem=send_sems.at[j],
                    recv_sem=recv_sems.at[j],
                    device_id=peer,
                    device_id_type=pl.DeviceIdType.MESH,
                ).wait_send()

    out_flat = pl.pallas_call(
        body,
        out_shape=jax.ShapeDtypeStruct((m * n,), x.dtype),
        in_specs=[
            pl.BlockSpec(memory_space=pltpu.VMEM),
            pl.BlockSpec(memory_space=pltpu.SMEM),
            pl.BlockSpec(memory_space=pltpu.SMEM),
        ],
        out_specs=pl.BlockSpec(memory_space=pltpu.VMEM),
        scratch_shapes=[
            pltpu.VMEM((m * n,), x.dtype),
            pltpu.SemaphoreType.DMA((m // CHUNK,)),
            pltpu.SemaphoreType.DMA((m // CHUNK,)),
        ],
        compiler_params=pltpu.CompilerParams(collective_id=0),
    )(x.reshape(m * n), src_of, counts)
    return out_flat.reshape(m, n)
